# baseline (speedup 1.0000x reference)
import functools

import jax
import jax.numpy as jnp
from jax import lax
from jax.experimental import pallas as pl
from jax.experimental.pallas import tpu as pltpu
from jax.experimental.pallas import tpu_sc as plsc

N_NODES = 100000
HIDDEN = 64
N_EDGES = 1600000
VOCAB_HALF = 500000

NC = 2
NS = 16

N_PAD = 102400
E_ROWS = 12544
E_PAD = E_ROWS * 128
ROWS_PER_TILE = E_ROWS // NS
SLAB_ROWS = 8
N_SLABS = ROWS_PER_TILE // SLAB_ROWS

N_CHUNKS = 4
CHUNK = N_PAD // N_CHUNKS
DUMMY = CHUNK
ACC_ROWS = CHUNK + 128


def _embed_gather_kernel(table_hbm, ids_hbm, out_hbm,
                         idx_v, pidx_v, prow_v, out_v, sem):
    cid = lax.axis_index("c")
    sid = lax.axis_index("s")
    wid = sid * NC + cid
    base = wid * 3200
    off8 = (wid * 25) // 8 * 8
    delta = wid * 25 - off8
    pltpu.sync_copy(ids_hbm.at[pl.ds(pl.multiple_of(off8, 8), 32)], idx_v)

    def prep(i, carry):
        for k in range(8):
            pidx_v[i, pl.ds(k * 16, 16)] = lax.shift_right_logical(
                idx_v[i, pl.ds(k * 16, 16)], 1)
        return carry
    lax.fori_loop(0, 32, prep, 0)

    def body(j, carry):
        pltpu.async_copy(table_hbm.at[pidx_v.at[delta + j]], prow_v,
                         sem).wait()

        def grp(g, carry2):
            par16 = (idx_v[delta + j, pl.ds(g * 16, 16)] & 1) * 64
            for rr in range(16):
                r = g * 16 + rr
                off = par16[rr]
                for k in range(4):
                    out_v[r, pl.ds(k * 16, 16)] = (
                        prow_v[r, pl.ds(off + k * 16, 16)])
            return carry2

        lax.fori_loop(0, 8, grp, 0)
        pltpu.sync_copy(
            out_v,
            out_hbm.at[pl.ds(pl.multiple_of(base + j * 128, 128), 128)])
        return carry

    lax.fori_loop(0, 25, body, 0)


def _degree_kernel(dst_hbm, out_hbm, acc, dst_v, zbuf, ones_v):
    cid = lax.axis_index("c")
    sid = lax.axis_index("s")

    def zb(i, carry):
        zbuf[pl.ds(i * 16, 16)] = jnp.zeros((16,), jnp.float32)
        return carry
    lax.fori_loop(0, 400, zb, 0)
    pltpu.sync_copy(zbuf, acc.at[pl.ds(pl.multiple_of(sid * 6400, 8), 6400)])
    for k in range(8):
        ones_v[pl.ds(k * 16, 16)] = jnp.full((16,), 1.0, jnp.float32)
    plsc.subcore_barrier()

    base = cid * (E_ROWS // 2) + sid * (ROWS_PER_TILE // 2)

    def body(s, carry):
        pltpu.sync_copy(
            dst_hbm.at[pl.ds(pl.multiple_of(base + s * SLAB_ROWS, 8),
                             SLAB_ROWS)], dst_v)
        for j in range(SLAB_ROWS):
            pltpu.sync_copy(ones_v, acc.at[dst_v.at[j]], add=True)
        return carry

    lax.fori_loop(0, N_SLABS // 2, body, 0)
    plsc.subcore_barrier()
    pltpu.sync_copy(
        acc.at[pl.ds(pl.multiple_of(sid * 6400, 8), 6400)],
        out_hbm.at[pl.ds(pl.multiple_of(cid * N_PAD + sid * 6400, 8), 6400)])


CW = 16
HS_ROWS = 4
N_HS = ROWS_PER_TILE // HS_ROWS


def _agg_kernel(h0_hbm, h1_hbm, h2_hbm, h3_hbm, src_hbm, dst_hbm, out_hbm,
                acc, esrc_v, edst_v, sidx_v, rows_v, zbuf_v,
                sem_g, sem_e0, sem_e1, sem_s0, sem_s1):
    cid = lax.axis_index("c")
    sid = lax.axis_index("s")
    sem_e = (sem_e0, sem_e1)
    sem_s = (sem_s0, sem_s1)

    def zb(i, carry):
        zbuf_v[i, pl.ds(0, CW)] = jnp.zeros((CW,), jnp.float32)
        return carry
    lax.fori_loop(0, 256, zb, 0)

    ebase = sid * ROWS_PER_TILE

    def stage(t, b):
        off = pl.multiple_of(ebase + t * HS_ROWS, 4)
        pltpu.async_copy(src_hbm.at[pl.ds(off, HS_ROWS)], esrc_v.at[b],
                         sem_e[b])
        pltpu.async_copy(dst_hbm.at[pl.ds(off, HS_ROWS)], edst_v.at[b],
                         sem_e[b])

    def wait_stage(b):
        pltpu.make_async_copy(src_hbm.at[pl.ds(0, HS_ROWS)], esrc_v.at[b],
                              sem_e[b]).wait()
        pltpu.make_async_copy(dst_hbm.at[pl.ds(0, HS_ROWS)], edst_v.at[b],
                              sem_e[b]).wait()

    def drain_scatter(b):
        for j in range(HS_ROWS):
            pltpu.make_async_copy(rows_v.at[b, pl.ds(j * 128, 128)],
                                  acc.at[sidx_v.at[b, j]], sem_s[b]).wait()

    def process(h_hbm, cc):
        def zc(z, carry):
            pltpu.sync_copy(
                zbuf_v,
                acc.at[pl.ds(pl.multiple_of(sid * 6400 + z * 256, 8), 256)])
            return carry
        lax.fori_loop(0, 25, zc, 0)
        plsc.subcore_barrier()

        stage(0, 0)
        stage(1, 1)

        def body(i, carry):
            for b in range(2):
                t = 2 * i + b
                @pl.when(t >= 2)
                def _():
                    drain_scatter(b)
                wait_stage(b)
                for j in range(HS_ROWS):
                    for k in range(8):
                        sidx_v[b, j, pl.ds(k * 16, 16)] = (
                            edst_v[b, j, pl.ds(k * 16, 16)])
                cps = [pltpu.async_copy(h_hbm.at[esrc_v.at[b, j]],
                                        rows_v.at[b, pl.ds(j * 128, 128)],
                                        sem_g)
                       for j in range(HS_ROWS)]
                for cp in cps:
                    cp.wait()
                for j in range(HS_ROWS):
                    pltpu.async_copy(rows_v.at[b, pl.ds(j * 128, 128)],
                                     acc.at[sidx_v.at[b, j]], sem_s[b],
                                     add=True)
                stage(jnp.minimum(t + 2, N_HS - 1), b)
            return carry

        lax.fori_loop(0, N_HS // 2, body, 0)
        for b in range(2):
            drain_scatter(b)
            wait_stage(b)
        plsc.subcore_barrier()

        def wc(z, carry):
            r = pl.multiple_of(sid * 6400 + z * 512, 8)
            pltpu.sync_copy(acc.at[pl.ds(r, 512)],
                            out_hbm.at[pl.ds(r, 512), pl.ds(cc * CW, CW)])
            return carry
        lax.fori_loop(0, 12, wc, 0)
        r = pl.multiple_of(sid * 6400 + 6144, 8)
        pltpu.sync_copy(acc.at[pl.ds(r, 256)],
                        out_hbm.at[pl.ds(r, 256), pl.ds(cc * CW, CW)])
        plsc.subcore_barrier()

    hs = [h0_hbm, h1_hbm, h2_hbm, h3_hbm]
    for chunk_i in range(2):
        for c in range(NC):
            cc = chunk_i * NC + c

            @pl.when(cid == c)
            def _():
                process(hs[cc], cc)


def _finish_kernel_body(with_agg, with_cols, refs):
    if with_agg:
        (h_hbm, agg_hbm, degp_hbm), rest = refs[:3], refs[3:]
    else:
        (h_hbm,), rest = refs[:1], refs[1:]
    if with_cols:
        outs, rest = rest[:5], rest[5:]
    else:
        outs, rest = rest[:1], rest[1:]
    hn_v, hc_v, hbuf_v, abuf_v, dbuf_v = rest
    cid = lax.axis_index("c")
    sid = lax.axis_index("s")
    wid = sid * NC + cid
    base = wid * 3200

    def blk(b, carry):
        r0 = pl.multiple_of(base + b * 128, 128)
        pltpu.sync_copy(h_hbm.at[pl.ds(r0, 128)], hbuf_v)
        if with_agg:
            pltpu.sync_copy(agg_hbm.at[pl.ds(r0, 128)], abuf_v)
            pltpu.sync_copy(degp_hbm.at[pl.ds(r0, 128)], dbuf_v.at[0])
            pltpu.sync_copy(degp_hbm.at[pl.ds(N_PAD + r0, 128)],
                            dbuf_v.at[1])

        def row(g, carry2):
            if with_agg:
                d16 = jnp.maximum(
                    dbuf_v[0, pl.ds(g * 16, 16)] + dbuf_v[1, pl.ds(g * 16, 16)],
                    1.0)
                rb = lax.bitcast_convert_type(d16, jnp.int32)
                rd16 = lax.bitcast_convert_type(
                    jnp.full((16,), 0x7EF311C3, jnp.int32) - rb, jnp.float32)
                for _ in range(3):
                    rd16 = rd16 * (2.0 - d16 * rd16)
            for rr in range(16):
                r = g * 16 + rr
                ys = []
                if with_agg:
                    rd = rd16[rr]
                ss = jnp.zeros((16,), jnp.float32)
                for k in range(4):
                    y = hbuf_v[r, pl.ds(k * 16, 16)]
                    if with_agg:
                        y = y + abuf_v[r, pl.ds(k * 16, 16)] * rd
                    ys.append(y)
                    ss = ss + y * y
                tot = jnp.maximum(jnp.sum(ss), 1e-24)
                bits = lax.bitcast_convert_type(tot, jnp.int32)
                g_ = lax.bitcast_convert_type(
                    jnp.int32(0x5F3759DF) - lax.shift_right_logical(bits, 1),
                    jnp.float32)
                for _ in range(3):
                    g_ = g_ * (1.5 - 0.5 * tot * g_ * g_)
                for k in range(4):
                    yk = ys[k] * g_
                    hn_v[r, pl.ds(k * 16, 16)] = yk
                    if with_cols:
                        hc_v[k, r, pl.ds(0, CW)] = yk
            return carry2

        lax.fori_loop(0, 8, row, 0)
        pltpu.sync_copy(hn_v, outs[0].at[pl.ds(r0, 128)])
        if with_cols:
            for k in range(4):
                pltpu.sync_copy(hc_v.at[k], outs[1 + k].at[pl.ds(r0, 128)])
        return carry

    lax.fori_loop(0, 25, blk, 0)


def _finish(h, agg=None, degp=None, with_cols=True):
    with_agg = agg is not None
    args = (h,) + ((agg, degp) if with_agg else ())

    def body(*refs):
        _finish_kernel_body(with_agg, with_cols, refs)

    out_type = [jax.ShapeDtypeStruct((N_PAD, HIDDEN), jnp.float32)]
    if with_cols:
        out_type += [jax.ShapeDtypeStruct((N_PAD, CW), jnp.float32)] * 4
    res = pl.kernel(
        body,
        out_type=out_type,
        mesh=_sc_mesh(),
        compiler_params=pltpu.CompilerParams(use_tc_tiling_on_sc=False,
                                             needs_layout_passes=False),
        scratch_types=[
            pltpu.VMEM((128, HIDDEN), jnp.float32),
            pltpu.VMEM((4, 128, CW), jnp.float32),
            pltpu.VMEM((128, HIDDEN), jnp.float32),
            pltpu.VMEM((128, HIDDEN), jnp.float32),
            pltpu.VMEM((2, 128), jnp.float32),
        ],
    )(*args)
    return res if with_cols else res[0]


def _sc_mesh():
    return plsc.VectorSubcoreMesh(core_axis_name="c", subcore_axis_name="s",
                                  num_cores=NC, num_subcores=NS)


def _embed_gather(table2, ids2d):
    return pl.kernel(
        _embed_gather_kernel,
        out_type=jax.ShapeDtypeStruct((N_PAD, HIDDEN), jnp.float32),
        mesh=_sc_mesh(),
        compiler_params=pltpu.CompilerParams(use_tc_tiling_on_sc=False,
                                             needs_layout_passes=False),
        scratch_types=[
            pltpu.VMEM((32, 128), jnp.int32),
            pltpu.VMEM((32, 128), jnp.int32),
            pltpu.VMEM((128, 128), jnp.float32),
            pltpu.VMEM((128, HIDDEN), jnp.float32),
            pltpu.SemaphoreType.DMA,
        ],
    )(table2, ids2d)


def _degree(dst2d):
    return pl.kernel(
        _degree_kernel,
        out_type=jax.ShapeDtypeStruct((NC * N_PAD,), jnp.float32),
        mesh=_sc_mesh(),
        compiler_params=pltpu.CompilerParams(use_tc_tiling_on_sc=False),
        scratch_types=[
            pltpu.VMEM_SHARED((N_PAD,), jnp.float32),
            pltpu.VMEM((SLAB_ROWS, 128), jnp.int32),
            pltpu.VMEM((6400,), jnp.float32),
            pltpu.VMEM((128,), jnp.float32),
        ],
    )(dst2d)


def _aggregate(hcols, src2d, dst2d):
    return pl.kernel(
        _agg_kernel,
        out_type=jax.ShapeDtypeStruct((N_PAD, HIDDEN), jnp.float32),
        mesh=_sc_mesh(),
        compiler_params=pltpu.CompilerParams(use_tc_tiling_on_sc=False),
        scratch_types=[
            pltpu.VMEM_SHARED((N_PAD, CW), jnp.float32),
            pltpu.VMEM((2, HS_ROWS, 128), jnp.int32),
            pltpu.VMEM((2, HS_ROWS, 128), jnp.int32),
            pltpu.VMEM((2, HS_ROWS, 128), jnp.int32),
            pltpu.VMEM((2, HS_ROWS * 128, CW), jnp.float32),
            pltpu.VMEM((256, CW), jnp.float32),
            pltpu.SemaphoreType.DMA,
            pltpu.SemaphoreType.DMA,
            pltpu.SemaphoreType.DMA,
            pltpu.SemaphoreType.DMA,
            pltpu.SemaphoreType.DMA,
        ],
    )(hcols[0], hcols[1], hcols[2], hcols[3], src2d, dst2d)


def _l2norm_body(x_ref, o_ref, c0_ref, c1_ref, c2_ref, c3_ref):
    x = x_ref[...]
    n = jnp.sqrt(jnp.sum(x * x, axis=-1, keepdims=True))
    y = x / jnp.maximum(n, 1e-12)
    o_ref[...] = y
    for c, ref in enumerate((c0_ref, c1_ref, c2_ref, c3_ref)):
        ref[...] = y[:, c * CW:(c + 1) * CW]


def _l2norm(x):
    return pl.pallas_call(
        _l2norm_body,
        out_shape=[jax.ShapeDtypeStruct((N_PAD, HIDDEN), jnp.float32)]
        + [jax.ShapeDtypeStruct((N_PAD, CW), jnp.float32)] * 4,
        grid=(N_PAD // 1024,),
        in_specs=[pl.BlockSpec((1024, HIDDEN), lambda i: (i, 0))],
        out_specs=[pl.BlockSpec((1024, HIDDEN), lambda i: (i, 0))]
        + [pl.BlockSpec((1024, CW), lambda i: (i, 0))] * 4,
    )(x)


def _hop_finish_body(h_ref, agg_ref, degp_ref, o_ref,
                     c0_ref, c1_ref, c2_ref, c3_ref):
    h = h_ref[...]
    deg = jnp.maximum(degp_ref[0, :] + degp_ref[1, :], 1.0)
    y = h + agg_ref[...] / deg[:, None]
    n = jnp.sqrt(jnp.sum(y * y, axis=-1, keepdims=True))
    y = y / jnp.maximum(n, 1e-12)
    o_ref[...] = y
    for c, ref in enumerate((c0_ref, c1_ref, c2_ref, c3_ref)):
        ref[...] = y[:, c * CW:(c + 1) * CW]


def _hop_finish(h, agg, degp):
    return pl.pallas_call(
        _hop_finish_body,
        out_shape=[jax.ShapeDtypeStruct((N_PAD, HIDDEN), jnp.float32)]
        + [jax.ShapeDtypeStruct((N_PAD, CW), jnp.float32)] * 4,
        grid=(N_PAD // 1024,),
        in_specs=[
            pl.BlockSpec((1024, HIDDEN), lambda i: (i, 0)),
            pl.BlockSpec((1024, HIDDEN), lambda i: (i, 0)),
            pl.BlockSpec((NC, 1024), lambda i: (0, i)),
        ],
        out_specs=[pl.BlockSpec((1024, HIDDEN), lambda i: (i, 0))]
        + [pl.BlockSpec((1024, CW), lambda i: (i, 0))] * 4,
    )(h, agg, degp)


def _decode_body(h_ref, w_ref, b_ref, o_ref):
    o_ref[...] = jnp.dot(h_ref[...], w_ref[...],
                         preferred_element_type=jnp.float32) + b_ref[...]


def _decode(h, w, b):
    return pl.pallas_call(
        _decode_body,
        out_shape=jax.ShapeDtypeStruct((N_PAD, HIDDEN), jnp.float32),
        grid=(N_PAD // 1024,),
        in_specs=[
            pl.BlockSpec((1024, HIDDEN), lambda i: (i, 0)),
            pl.BlockSpec((HIDDEN, HIDDEN), lambda i: (0, 0)),
            pl.BlockSpec((1, HIDDEN), lambda i: (0, 0)),
        ],
        out_specs=pl.BlockSpec((1024, HIDDEN), lambda i: (i, 0)),
    )(h, w, b.reshape(1, HIDDEN))


@jax.jit
def kernel(node_ids, edge_index, embed_table, dec_W, dec_b):
    ids2d = jnp.pad(node_ids.astype(jnp.int32),
                    (0, N_PAD - N_NODES)).reshape(N_PAD // 128, 128)
    src = jnp.pad(edge_index[0], (0, E_PAD - N_EDGES)).reshape(E_ROWS, 128)
    dst = jnp.pad(edge_index[1], (0, E_PAD - N_EDGES),
                  constant_values=N_NODES).reshape(E_ROWS, 128)

    table2 = embed_table.reshape(VOCAB_HALF, 128)
    x = _embed_gather(table2, ids2d)
    h, *hcols = _finish(x)
    degp = _degree(dst)
    agg = _aggregate(hcols, src, dst)
    h, *hcols = _finish(h, agg, degp)
    agg = _aggregate(hcols, src, dst)
    h = _finish(h, agg, degp, with_cols=False)
    out = _decode(h, dec_W, dec_b)
    return out[:N_NODES]


# speedup vs baseline: 1.0913x; 1.0913x over previous
import functools

import jax
import jax.numpy as jnp
from jax import lax
from jax.experimental import pallas as pl
from jax.experimental.pallas import tpu as pltpu
from jax.experimental.pallas import tpu_sc as plsc

N_NODES = 100000
HIDDEN = 64
N_EDGES = 1600000
VOCAB_HALF = 500000

NC = 2
NS = 16

N_PAD = 102400
E_ROWS = 12544
E_PAD = E_ROWS * 128
ROWS_PER_TILE = E_ROWS // NS
SLAB_ROWS = 8
N_SLABS = ROWS_PER_TILE // SLAB_ROWS

N_CHUNKS = 4
CHUNK = N_PAD // N_CHUNKS
DUMMY = CHUNK
ACC_ROWS = CHUNK + 128


def _embed_gather_kernel(table_hbm, ids_hbm, out_hbm, idx_v, rows_v, sem):
    cid = lax.axis_index("c")
    sid = lax.axis_index("s")
    wid = sid * NC + cid
    base = wid * 3200
    off8 = (wid * 25) // 8 * 8
    delta = wid * 25 - off8
    pltpu.sync_copy(ids_hbm.at[pl.ds(pl.multiple_of(off8, 8), 32)], idx_v)

    def body(j, carry):
        pltpu.async_copy(table_hbm.at[idx_v.at[delta + j]], rows_v,
                         sem).wait()
        pltpu.sync_copy(
            rows_v,
            out_hbm.at[pl.ds(pl.multiple_of(base + j * 128, 128), 128)])
        return carry

    lax.fori_loop(0, 25, body, 0)


def _degree_kernel(dst_hbm, out_hbm, acc, dst_v, zbuf, ones_v):
    cid = lax.axis_index("c")
    sid = lax.axis_index("s")

    def zb(i, carry):
        zbuf[pl.ds(i * 16, 16)] = jnp.zeros((16,), jnp.float32)
        return carry
    lax.fori_loop(0, 400, zb, 0)
    pltpu.sync_copy(zbuf, acc.at[pl.ds(pl.multiple_of(sid * 6400, 8), 6400)])
    for k in range(8):
        ones_v[pl.ds(k * 16, 16)] = jnp.full((16,), 1.0, jnp.float32)
    plsc.subcore_barrier()

    base = cid * (E_ROWS // 2) + sid * (ROWS_PER_TILE // 2)

    def body(s, carry):
        pltpu.sync_copy(
            dst_hbm.at[pl.ds(pl.multiple_of(base + s * SLAB_ROWS, 8),
                             SLAB_ROWS)], dst_v)
        for j in range(SLAB_ROWS):
            pltpu.sync_copy(ones_v, acc.at[dst_v.at[j]], add=True)
        return carry

    lax.fori_loop(0, N_SLABS // 2, body, 0)
    plsc.subcore_barrier()
    pltpu.sync_copy(
        acc.at[pl.ds(pl.multiple_of(sid * 6400, 8), 6400)],
        out_hbm.at[pl.ds(pl.multiple_of(cid * N_PAD + sid * 6400, 8), 6400)])


CW = 16
HS_ROWS = 4
N_HS = ROWS_PER_TILE // HS_ROWS


def _agg_kernel(h0_hbm, h1_hbm, h2_hbm, h3_hbm, src_hbm, dst_hbm, out_hbm,
                acc, esrc_v, edst_v, sidx_v, rows_v, zbuf_v,
                sem_g, sem_e0, sem_e1, sem_s0, sem_s1):
    cid = lax.axis_index("c")
    sid = lax.axis_index("s")
    sem_e = (sem_e0, sem_e1)
    sem_s = (sem_s0, sem_s1)

    def zb(i, carry):
        zbuf_v[i, pl.ds(0, CW)] = jnp.zeros((CW,), jnp.float32)
        return carry
    lax.fori_loop(0, 256, zb, 0)

    ebase = sid * ROWS_PER_TILE

    def stage(t, b):
        off = pl.multiple_of(ebase + t * HS_ROWS, 4)
        pltpu.async_copy(src_hbm.at[pl.ds(off, HS_ROWS)], esrc_v.at[b],
                         sem_e[b])
        pltpu.async_copy(dst_hbm.at[pl.ds(off, HS_ROWS)], edst_v.at[b],
                         sem_e[b])

    def wait_stage(b):
        pltpu.make_async_copy(src_hbm.at[pl.ds(0, HS_ROWS)], esrc_v.at[b],
                              sem_e[b]).wait()
        pltpu.make_async_copy(dst_hbm.at[pl.ds(0, HS_ROWS)], edst_v.at[b],
                              sem_e[b]).wait()

    def drain_scatter(b):
        for j in range(HS_ROWS):
            pltpu.make_async_copy(rows_v.at[b, pl.ds(j * 128, 128)],
                                  acc.at[sidx_v.at[b, j]], sem_s[b]).wait()

    def process(h_hbm, cc):
        def zc(z, carry):
            pltpu.sync_copy(
                zbuf_v,
                acc.at[pl.ds(pl.multiple_of(sid * 6400 + z * 256, 8), 256)])
            return carry
        lax.fori_loop(0, 25, zc, 0)
        plsc.subcore_barrier()

        stage(0, 0)
        stage(1, 1)

        def body(i, carry):
            for b in range(2):
                t = 2 * i + b
                @pl.when(t >= 2)
                def _():
                    drain_scatter(b)
                wait_stage(b)
                for j in range(HS_ROWS):
                    for k in range(8):
                        sidx_v[b, j, pl.ds(k * 16, 16)] = (
                            edst_v[b, j, pl.ds(k * 16, 16)])
                cps = [pltpu.async_copy(h_hbm.at[esrc_v.at[b, j]],
                                        rows_v.at[b, pl.ds(j * 128, 128)],
                                        sem_g)
                       for j in range(HS_ROWS)]
                for cp in cps:
                    cp.wait()
                for j in range(HS_ROWS):
                    pltpu.async_copy(rows_v.at[b, pl.ds(j * 128, 128)],
                                     acc.at[sidx_v.at[b, j]], sem_s[b],
                                     add=True)
                stage(jnp.minimum(t + 2, N_HS - 1), b)
            return carry

        lax.fori_loop(0, N_HS // 2, body, 0)
        for b in range(2):
            drain_scatter(b)
            wait_stage(b)
        plsc.subcore_barrier()

        def wc(z, carry):
            r = pl.multiple_of(sid * 6400 + z * 512, 8)
            pltpu.sync_copy(acc.at[pl.ds(r, 512)],
                            out_hbm.at[pl.ds(r, 512), pl.ds(cc * CW, CW)])
            return carry
        lax.fori_loop(0, 12, wc, 0)
        r = pl.multiple_of(sid * 6400 + 6144, 8)
        pltpu.sync_copy(acc.at[pl.ds(r, 256)],
                        out_hbm.at[pl.ds(r, 256), pl.ds(cc * CW, CW)])
        plsc.subcore_barrier()

    hs = [h0_hbm, h1_hbm, h2_hbm, h3_hbm]
    for chunk_i in range(2):
        for c in range(NC):
            cc = chunk_i * NC + c

            @pl.when(cid == c)
            def _():
                process(hs[cc], cc)


def _finish_kernel_body(with_agg, with_cols, refs):
    if with_agg:
        (h_hbm, agg_hbm, degp_hbm), rest = refs[:3], refs[3:]
    else:
        (h_hbm,), rest = refs[:1], refs[1:]
    if with_cols:
        outs, rest = rest[:5], rest[5:]
    else:
        outs, rest = rest[:1], rest[1:]
    hn_v, hc_v, hbuf_v, abuf_v, dbuf_v, sem_i, sem_o = rest
    cid = lax.axis_index("c")
    sid = lax.axis_index("s")
    wid = sid * NC + cid
    base = wid * 3200

    def drain_out(outs):
        pltpu.make_async_copy(hn_v, outs[0].at[pl.ds(0, 128)], sem_o).wait()
        if with_cols:
            for k in range(4):
                pltpu.make_async_copy(hc_v.at[k], outs[1 + k].at[pl.ds(0, 128)],
                                      sem_o).wait()

    def blk(b, carry):
        r0 = pl.multiple_of(base + b * 128, 128)
        cps = [pltpu.async_copy(h_hbm.at[pl.ds(r0, 128)], hbuf_v, sem_i)]
        if with_agg:
            cps.append(pltpu.async_copy(agg_hbm.at[pl.ds(r0, 128)], abuf_v,
                                        sem_i))
            cps.append(pltpu.async_copy(degp_hbm.at[pl.ds(r0, 128)],
                                        dbuf_v.at[0], sem_i))
            cps.append(pltpu.async_copy(degp_hbm.at[pl.ds(N_PAD + r0, 128)],
                                        dbuf_v.at[1], sem_i))
        for cp in cps:
            cp.wait()

        @pl.when(b >= 1)
        def _():
            drain_out(outs)

        def row(g, carry2):
            if with_agg:
                d16 = jnp.maximum(
                    dbuf_v[0, pl.ds(g * 16, 16)] + dbuf_v[1, pl.ds(g * 16, 16)],
                    1.0)
                rb = lax.bitcast_convert_type(d16, jnp.int32)
                rd16 = lax.bitcast_convert_type(
                    jnp.full((16,), 0x7EF311C3, jnp.int32) - rb, jnp.float32)
                for _ in range(3):
                    rd16 = rd16 * (2.0 - d16 * rd16)
            for rr in range(16):
                r = g * 16 + rr
                ys = []
                if with_agg:
                    rd = rd16[rr]
                ss = jnp.zeros((16,), jnp.float32)
                for k in range(4):
                    y = hbuf_v[r, pl.ds(k * 16, 16)]
                    if with_agg:
                        y = y + abuf_v[r, pl.ds(k * 16, 16)] * rd
                    ys.append(y)
                    ss = ss + y * y
                tot = jnp.maximum(jnp.sum(ss), 1e-24)
                bits = lax.bitcast_convert_type(tot, jnp.int32)
                g_ = lax.bitcast_convert_type(
                    jnp.int32(0x5F3759DF) - lax.shift_right_logical(bits, 1),
                    jnp.float32)
                for _ in range(3):
                    g_ = g_ * (1.5 - 0.5 * tot * g_ * g_)
                for k in range(4):
                    yk = ys[k] * g_
                    hn_v[r, pl.ds(k * 16, 16)] = yk
                    if with_cols:
                        hc_v[k, r, pl.ds(0, CW)] = yk
            return carry2

        lax.fori_loop(0, 8, row, 0)
        pltpu.async_copy(hn_v, outs[0].at[pl.ds(r0, 128)], sem_o)
        if with_cols:
            for k in range(4):
                pltpu.async_copy(hc_v.at[k], outs[1 + k].at[pl.ds(r0, 128)],
                                 sem_o)
        return carry

    lax.fori_loop(0, 25, blk, 0)
    drain_out(outs)


def _finish(h, agg=None, degp=None, with_cols=True):
    with_agg = agg is not None
    args = (h,) + ((agg, degp) if with_agg else ())

    def body(*refs):
        _finish_kernel_body(with_agg, with_cols, refs)

    out_type = [jax.ShapeDtypeStruct((N_PAD, HIDDEN), jnp.float32)]
    if with_cols:
        out_type += [jax.ShapeDtypeStruct((N_PAD, CW), jnp.float32)] * 4
    res = pl.kernel(
        body,
        out_type=out_type,
        mesh=_sc_mesh(),
        compiler_params=pltpu.CompilerParams(use_tc_tiling_on_sc=False,
                                             needs_layout_passes=False),
        scratch_types=[
            pltpu.VMEM((128, HIDDEN), jnp.float32),
            pltpu.VMEM((4, 128, CW), jnp.float32),
            pltpu.VMEM((128, HIDDEN), jnp.float32),
            pltpu.VMEM((128, HIDDEN), jnp.float32),
            pltpu.VMEM((2, 128), jnp.float32),
            pltpu.SemaphoreType.DMA,
            pltpu.SemaphoreType.DMA,
        ],
    )(*args)
    return res if with_cols else res[0]


def _sc_mesh():
    return plsc.VectorSubcoreMesh(core_axis_name="c", subcore_axis_name="s",
                                  num_cores=NC, num_subcores=NS)


def _embed_gather(table, ids2d):
    return pl.kernel(
        _embed_gather_kernel,
        out_type=jax.ShapeDtypeStruct((N_PAD, HIDDEN), jnp.float32),
        mesh=_sc_mesh(),
        compiler_params=pltpu.CompilerParams(use_tc_tiling_on_sc=False),
        scratch_types=[
            pltpu.VMEM((32, 128), jnp.int32),
            pltpu.VMEM((128, HIDDEN), jnp.float32),
            pltpu.SemaphoreType.DMA,
        ],
    )(table, ids2d)


def _degree(dst2d):
    return pl.kernel(
        _degree_kernel,
        out_type=jax.ShapeDtypeStruct((NC * N_PAD,), jnp.float32),
        mesh=_sc_mesh(),
        compiler_params=pltpu.CompilerParams(use_tc_tiling_on_sc=False),
        scratch_types=[
            pltpu.VMEM_SHARED((N_PAD,), jnp.float32),
            pltpu.VMEM((SLAB_ROWS, 128), jnp.int32),
            pltpu.VMEM((6400,), jnp.float32),
            pltpu.VMEM((128,), jnp.float32),
        ],
    )(dst2d)


def _aggregate(hcols, src2d, dst2d):
    return pl.kernel(
        _agg_kernel,
        out_type=jax.ShapeDtypeStruct((N_PAD, HIDDEN), jnp.float32),
        mesh=_sc_mesh(),
        compiler_params=pltpu.CompilerParams(use_tc_tiling_on_sc=False),
        scratch_types=[
            pltpu.VMEM_SHARED((N_PAD, CW), jnp.float32),
            pltpu.VMEM((2, HS_ROWS, 128), jnp.int32),
            pltpu.VMEM((2, HS_ROWS, 128), jnp.int32),
            pltpu.VMEM((2, HS_ROWS, 128), jnp.int32),
            pltpu.VMEM((2, HS_ROWS * 128, CW), jnp.float32),
            pltpu.VMEM((256, CW), jnp.float32),
            pltpu.SemaphoreType.DMA,
            pltpu.SemaphoreType.DMA,
            pltpu.SemaphoreType.DMA,
            pltpu.SemaphoreType.DMA,
            pltpu.SemaphoreType.DMA,
        ],
    )(hcols[0], hcols[1], hcols[2], hcols[3], src2d, dst2d)


def _l2norm_body(x_ref, o_ref, c0_ref, c1_ref, c2_ref, c3_ref):
    x = x_ref[...]
    n = jnp.sqrt(jnp.sum(x * x, axis=-1, keepdims=True))
    y = x / jnp.maximum(n, 1e-12)
    o_ref[...] = y
    for c, ref in enumerate((c0_ref, c1_ref, c2_ref, c3_ref)):
        ref[...] = y[:, c * CW:(c + 1) * CW]


def _l2norm(x):
    return pl.pallas_call(
        _l2norm_body,
        out_shape=[jax.ShapeDtypeStruct((N_PAD, HIDDEN), jnp.float32)]
        + [jax.ShapeDtypeStruct((N_PAD, CW), jnp.float32)] * 4,
        grid=(N_PAD // 1024,),
        in_specs=[pl.BlockSpec((1024, HIDDEN), lambda i: (i, 0))],
        out_specs=[pl.BlockSpec((1024, HIDDEN), lambda i: (i, 0))]
        + [pl.BlockSpec((1024, CW), lambda i: (i, 0))] * 4,
    )(x)


def _hop_finish_body(h_ref, agg_ref, degp_ref, o_ref,
                     c0_ref, c1_ref, c2_ref, c3_ref):
    h = h_ref[...]
    deg = jnp.maximum(degp_ref[0, :] + degp_ref[1, :], 1.0)
    y = h + agg_ref[...] / deg[:, None]
    n = jnp.sqrt(jnp.sum(y * y, axis=-1, keepdims=True))
    y = y / jnp.maximum(n, 1e-12)
    o_ref[...] = y
    for c, ref in enumerate((c0_ref, c1_ref, c2_ref, c3_ref)):
        ref[...] = y[:, c * CW:(c + 1) * CW]


def _hop_finish(h, agg, degp):
    return pl.pallas_call(
        _hop_finish_body,
        out_shape=[jax.ShapeDtypeStruct((N_PAD, HIDDEN), jnp.float32)]
        + [jax.ShapeDtypeStruct((N_PAD, CW), jnp.float32)] * 4,
        grid=(N_PAD // 1024,),
        in_specs=[
            pl.BlockSpec((1024, HIDDEN), lambda i: (i, 0)),
            pl.BlockSpec((1024, HIDDEN), lambda i: (i, 0)),
            pl.BlockSpec((NC, 1024), lambda i: (0, i)),
        ],
        out_specs=[pl.BlockSpec((1024, HIDDEN), lambda i: (i, 0))]
        + [pl.BlockSpec((1024, CW), lambda i: (i, 0))] * 4,
    )(h, agg, degp)


def _decode_body(h_ref, w_ref, b_ref, o_ref):
    o_ref[...] = jnp.dot(h_ref[...], w_ref[...],
                         preferred_element_type=jnp.float32) + b_ref[...]


def _decode(h, w, b):
    return pl.pallas_call(
        _decode_body,
        out_shape=jax.ShapeDtypeStruct((N_PAD, HIDDEN), jnp.float32),
        grid=(N_PAD // 1024,),
        in_specs=[
            pl.BlockSpec((1024, HIDDEN), lambda i: (i, 0)),
            pl.BlockSpec((HIDDEN, HIDDEN), lambda i: (0, 0)),
            pl.BlockSpec((1, HIDDEN), lambda i: (0, 0)),
        ],
        out_specs=pl.BlockSpec((1024, HIDDEN), lambda i: (i, 0)),
    )(h, w, b.reshape(1, HIDDEN))


@jax.jit
def kernel(node_ids, edge_index, embed_table, dec_W, dec_b):
    ids2d = jnp.pad(node_ids.astype(jnp.int32),
                    (0, N_PAD - N_NODES)).reshape(N_PAD // 128, 128)
    src = jnp.pad(edge_index[0], (0, E_PAD - N_EDGES)).reshape(E_ROWS, 128)
    dst = jnp.pad(edge_index[1], (0, E_PAD - N_EDGES),
                  constant_values=N_NODES).reshape(E_ROWS, 128)

    x = _embed_gather(embed_table, ids2d)
    h, *hcols = _finish(x)
    degp = _degree(dst)
    agg = _aggregate(hcols, src, dst)
    h, *hcols = _finish(h, agg, degp)
    agg = _aggregate(hcols, src, dst)
    h = _finish(h, agg, degp, with_cols=False)
    out = _decode(h, dec_W, dec_b)
    return out[:N_NODES]
